# Initial kernel scaffold; baseline (speedup 1.0000x reference)
import jax
import jax.numpy as jnp
from jax.experimental import pallas as pl


def kernel(x):
    raise NotImplementedError("write your pallas kernel here")


# speedup vs baseline: 13.8149x; 13.8149x over previous
import functools

import jax
import jax.numpy as jnp
from jax.experimental import pallas as pl

_TEMPERATURE = 10.0
_N = 32768
_ROWS = 128
_K = 1638
_BLOCK_ROWS = 8

_GUMBEL_U = jax.random.uniform(
    jax.random.key(42), (_ROWS, _N), dtype=jnp.float32, minval=1e-20, maxval=1.0
)
_GUMBEL = -jnp.log(-jnp.log(_GUMBEL_U))


def _kwinners_block(x_ref, g_ref, o_ref):
    x = x_ref[...]
    g = g_ref[...]
    noisy = x * (1.0 / _TEMPERATURE) + g
    bits = jax.lax.bitcast_convert_type(noisy, jnp.int32)
    ordk = jnp.where(bits >= 0, bits, bits ^ jnp.int32(0x7FFFFFFF))

    def body(i, prefix):
        bit = jnp.int32(31) - i
        cand = prefix + (jnp.int32(1) << bit)
        cnt = jnp.sum((ordk >= cand[:, None]).astype(jnp.int32), axis=1)
        return jnp.where(cnt >= _K, cand, prefix)

    init = jnp.full((x.shape[0],), jnp.iinfo(jnp.int32).min, jnp.int32)
    thresh = jax.lax.fori_loop(0, 32, body, init)
    o_ref[...] = jnp.where(ordk >= thresh[:, None], x, 0.0)


@functools.partial(jax.jit)
def kernel(x):
    grid = _ROWS // _BLOCK_ROWS
    spec = pl.BlockSpec((_BLOCK_ROWS, _N), lambda i: (i, 0))
    return pl.pallas_call(
        _kwinners_block,
        grid=(grid,),
        in_specs=[spec, spec],
        out_specs=spec,
        out_shape=jax.ShapeDtypeStruct((_ROWS, _N), jnp.float32),
    )(x, _GUMBEL)


# speedup vs baseline: 14.6105x; 1.0576x over previous
import functools

import jax
import jax.numpy as jnp
import numpy as np
from jax.experimental import pallas as pl

_TEMPERATURE = 10.0
_N = 32768
_ROWS = 128
_K = 1638
_BLOCK_ROWS = 32


def _threefry2x32(k0, k1, x0, x1):
    def rotl(x, d):
        return ((x << np.uint32(d)) | (x >> np.uint32(32 - d))).astype(np.uint32)

    k0 = np.uint32(k0)
    k1 = np.uint32(k1)
    ks2 = np.uint32(k0 ^ k1 ^ np.uint32(0x1BD11BDA))
    x0 = (x0 + k0).astype(np.uint32)
    x1 = (x1 + k1).astype(np.uint32)
    rot = [[13, 15, 26, 6], [17, 29, 16, 24]]
    keys = [(k1, ks2), (ks2, k0), (k0, k1), (k1, ks2), (ks2, k0)]
    for i in range(5):
        for d in rot[i % 2]:
            x0 = (x0 + x1).astype(np.uint32)
            x1 = rotl(x1, d)
            x1 = (x1 ^ x0).astype(np.uint32)
        a, b = keys[i]
        x0 = (x0 + a).astype(np.uint32)
        x1 = (x1 + b + np.uint32(i + 1)).astype(np.uint32)
    return x0, x1


def _gumbel_const():
    n = _ROWS * _N
    idx = np.arange(n, dtype=np.uint64)
    hi = (idx >> np.uint64(32)).astype(np.uint32)
    lo = (idx & np.uint64(0xFFFFFFFF)).astype(np.uint32)
    o0, o1 = _threefry2x32(0, 42, hi, lo)
    bits = (o0 ^ o1).reshape(_ROWS, _N)
    f = ((bits >> np.uint32(9)) | np.uint32(0x3F800000)).view(np.float32)
    f = f - np.float32(1.0)
    minval, maxval = np.float32(1e-20), np.float32(1.0)
    u = np.maximum(minval, f * (maxval - minval) + minval)
    return (-np.log(-np.log(u.astype(np.float64)))).astype(np.float32)


_GUMBEL = _gumbel_const()


def _ord_to_f32(ordk):
    bits = jnp.where(ordk >= 0, ordk, ordk ^ jnp.int32(0x7FFFFFFF))
    return jax.lax.bitcast_convert_type(bits, jnp.float32)


def _kwinners_block(x_ref, g_ref, o_ref):
    x = x_ref[...]
    g = g_ref[...]
    noisy = x * (1.0 / _TEMPERATURE) + g
    rows = x.shape[0]
    ones = jnp.ones((_N, 1), jnp.float32)

    def body(i, prefix_o):
        bit = jnp.int32(31) - i
        cand_o = prefix_o | (jnp.int32(1) << bit)
        cf = _ord_to_f32(cand_o ^ jnp.int32(-(2**31)))
        mask = (noisy >= cf).astype(jnp.float32)
        cnt = jax.lax.dot_general(
            mask, ones, (((1,), (0,)), ((), ())),
            preferred_element_type=jnp.float32,
        )
        return jnp.where(cnt >= _K, cand_o, prefix_o)

    init = jnp.zeros((rows, 1), jnp.int32)
    t_o = jax.lax.fori_loop(0, 32, body, init)
    tf = _ord_to_f32(t_o ^ jnp.int32(-(2**31)))
    o_ref[...] = jnp.where(noisy >= tf, x, 0.0)


@functools.partial(jax.jit)
def kernel(x):
    grid = _ROWS // _BLOCK_ROWS
    spec = pl.BlockSpec((_BLOCK_ROWS, _N), lambda i: (i, 0))
    return pl.pallas_call(
        _kwinners_block,
        grid=(grid,),
        in_specs=[spec, spec],
        out_specs=spec,
        out_shape=jax.ShapeDtypeStruct((_ROWS, _N), jnp.float32),
    )(x, _GUMBEL)


# speedup vs baseline: 33.1194x; 2.2668x over previous
import functools

import jax
import jax.numpy as jnp
import numpy as np
from jax.experimental import pallas as pl

_TEMPERATURE = 10.0
_N = 32768
_ROWS = 128
_K = 1638
_BLOCK_ROWS = 32


def _threefry2x32(k0, k1, x0, x1):
    def rotl(x, d):
        return ((x << np.uint32(d)) | (x >> np.uint32(32 - d))).astype(np.uint32)

    k0 = np.uint32(k0)
    k1 = np.uint32(k1)
    ks2 = np.uint32(k0 ^ k1 ^ np.uint32(0x1BD11BDA))
    x0 = (x0 + k0).astype(np.uint32)
    x1 = (x1 + k1).astype(np.uint32)
    rot = [[13, 15, 26, 6], [17, 29, 16, 24]]
    keys = [(k1, ks2), (ks2, k0), (k0, k1), (k1, ks2), (ks2, k0)]
    for i in range(5):
        for d in rot[i % 2]:
            x0 = (x0 + x1).astype(np.uint32)
            x1 = rotl(x1, d)
            x1 = (x1 ^ x0).astype(np.uint32)
        a, b = keys[i]
        x0 = (x0 + a).astype(np.uint32)
        x1 = (x1 + b + np.uint32(i + 1)).astype(np.uint32)
    return x0, x1


def _gumbel_const():
    n = _ROWS * _N
    idx = np.arange(n, dtype=np.uint64)
    hi = (idx >> np.uint64(32)).astype(np.uint32)
    lo = (idx & np.uint64(0xFFFFFFFF)).astype(np.uint32)
    o0, o1 = _threefry2x32(0, 42, hi, lo)
    bits = (o0 ^ o1).reshape(_ROWS, _N)
    f = ((bits >> np.uint32(9)) | np.uint32(0x3F800000)).view(np.float32)
    f = f - np.float32(1.0)
    minval, maxval = np.float32(1e-20), np.float32(1.0)
    u = np.maximum(minval, f * (maxval - minval) + minval)
    return (-np.log(-np.log(u.astype(np.float64)))).astype(np.float32)


_GUMBEL = _gumbel_const()
_GUMBEL_K = np.partition(_GUMBEL, _N - _K, axis=1)[:, _N - _K].reshape(_ROWS, 1)


def _f32_to_off(v):
    b = jax.lax.bitcast_convert_type(v, jnp.int32)
    ordk = jnp.where(b >= 0, b, b ^ jnp.int32(0x7FFFFFFF))
    return ordk ^ jnp.int32(-(2**31))


def _off_to_f32(o):
    ordk = o ^ jnp.int32(-(2**31))
    bits = jnp.where(ordk >= 0, ordk, ordk ^ jnp.int32(0x7FFFFFFF))
    return jax.lax.bitcast_convert_type(bits, jnp.float32)


def _kwinners_block(x_ref, g_ref, gk_ref, o_ref):
    x = x_ref[...]
    g = g_ref[...]
    gk = gk_ref[...]
    noisy = x * (1.0 / _TEMPERATURE) + g
    rows = x.shape[0]

    m = jnp.max(jnp.abs(x), axis=1, keepdims=True) * (1.0 / _TEMPERATURE)
    lo = gk - m - 1e-3
    hi = gk + m + 1e-3
    o_lo = _f32_to_off(lo)
    o_hi = _f32_to_off(hi)

    z = o_lo ^ o_hi
    zf = jnp.maximum(z, 1).astype(jnp.float32)
    zexp = (jax.lax.bitcast_convert_type(zf, jnp.int32) >> 23) - 127
    start = jnp.where(z < 0, jnp.int32(31), zexp.astype(jnp.int32))
    prefix0 = jnp.where(
        start >= 31, jnp.int32(0), o_lo & ~((jnp.int32(1) << (start + 1)) - 1)
    )
    nbits = jnp.max(start) + 1

    def body(i, prefix_o):
        bit = nbits - 1 - i
        cand_o = prefix_o | (jnp.int32(1) << bit)
        cf = _off_to_f32(cand_o)
        ones = jnp.where(noisy >= cf, jnp.int32(1), jnp.int32(0))
        cnt = jnp.sum(ones, axis=1, keepdims=True)
        return jnp.where(cnt >= _K, cand_o, prefix_o)

    t_o = jax.lax.fori_loop(0, nbits, body, prefix0)
    tf = _off_to_f32(t_o)
    o_ref[...] = jnp.where(noisy >= tf, x, 0.0)


@functools.partial(jax.jit)
def kernel(x):
    grid = _ROWS // _BLOCK_ROWS
    spec = pl.BlockSpec((_BLOCK_ROWS, _N), lambda i: (i, 0))
    kspec = pl.BlockSpec((_BLOCK_ROWS, 1), lambda i: (i, 0))
    return pl.pallas_call(
        _kwinners_block,
        grid=(grid,),
        in_specs=[spec, spec, kspec],
        out_specs=spec,
        out_shape=jax.ShapeDtypeStruct((_ROWS, _N), jnp.float32),
    )(x, _GUMBEL, _GUMBEL_K)


# speedup vs baseline: 40.5447x; 1.2242x over previous
import functools

import jax
import jax.numpy as jnp
import numpy as np
from jax.experimental import pallas as pl

_TEMPERATURE = 10.0
_N = 32768
_ROWS = 128
_K = 1638
_BLOCK_ROWS = 32


def _threefry2x32(k0, k1, x0, x1):
    def rotl(x, d):
        return ((x << np.uint32(d)) | (x >> np.uint32(32 - d))).astype(np.uint32)

    k0 = np.uint32(k0)
    k1 = np.uint32(k1)
    ks2 = np.uint32(k0 ^ k1 ^ np.uint32(0x1BD11BDA))
    x0 = (x0 + k0).astype(np.uint32)
    x1 = (x1 + k1).astype(np.uint32)
    rot = [[13, 15, 26, 6], [17, 29, 16, 24]]
    keys = [(k1, ks2), (ks2, k0), (k0, k1), (k1, ks2), (ks2, k0)]
    for i in range(5):
        for d in rot[i % 2]:
            x0 = (x0 + x1).astype(np.uint32)
            x1 = rotl(x1, d)
            x1 = (x1 ^ x0).astype(np.uint32)
        a, b = keys[i]
        x0 = (x0 + a).astype(np.uint32)
        x1 = (x1 + b + np.uint32(i + 1)).astype(np.uint32)
    return x0, x1


def _gumbel_const():
    n = _ROWS * _N
    idx = np.arange(n, dtype=np.uint64)
    hi = (idx >> np.uint64(32)).astype(np.uint32)
    lo = (idx & np.uint64(0xFFFFFFFF)).astype(np.uint32)
    o0, o1 = _threefry2x32(0, 42, hi, lo)
    bits = (o0 ^ o1).reshape(_ROWS, _N)
    f = ((bits >> np.uint32(9)) | np.uint32(0x3F800000)).view(np.float32)
    f = f - np.float32(1.0)
    minval, maxval = np.float32(1e-20), np.float32(1.0)
    u = np.maximum(minval, f * (maxval - minval) + minval)
    return (-np.log(-np.log(u.astype(np.float64)))).astype(np.float32)


_GUMBEL = _gumbel_const()
_GUMBEL_K = np.partition(_GUMBEL, _N - _K, axis=1)[:, _N - _K].reshape(_ROWS, 1)


def _f32_to_off(v):
    b = jax.lax.bitcast_convert_type(v, jnp.int32)
    ordk = jnp.where(b >= 0, b, b ^ jnp.int32(0x7FFFFFFF))
    return ordk ^ jnp.int32(-(2**31))


def _off_to_f32(o):
    ordk = o ^ jnp.int32(-(2**31))
    bits = jnp.where(ordk >= 0, ordk, ordk ^ jnp.int32(0x7FFFFFFF))
    return jax.lax.bitcast_convert_type(bits, jnp.float32)


def _kwinners_block(x_ref, g_ref, gk_ref, o_ref):
    x = x_ref[...]
    g = g_ref[...]
    gk = gk_ref[...]
    noisy = x * (1.0 / _TEMPERATURE) + g
    rows = x.shape[0]

    m = jnp.max(jnp.abs(x), axis=1, keepdims=True) * (1.0 / _TEMPERATURE)
    lo = gk - m - 1e-3
    hi = gk + m + 1e-3
    o_lo = _f32_to_off(lo)
    o_hi = _f32_to_off(hi)

    z = o_lo ^ o_hi
    zf = jnp.maximum(z, 1).astype(jnp.float32)
    zexp = (jax.lax.bitcast_convert_type(zf, jnp.int32) >> 23) - 127
    start = jnp.where(z < 0, jnp.int32(31), zexp.astype(jnp.int32))
    prefix0 = jnp.where(
        start >= 31, jnp.int32(0), o_lo & ~((jnp.int32(1) << (start + 1)) - 1)
    )
    nbits = jnp.max(start) + 1

    def cond(state):
        i, prefix_o, hit, hit_cand = state
        return jnp.logical_and(i < nbits, jnp.sum(hit) < rows)

    def body(state):
        i, prefix_o, hit, hit_cand = state
        bit = nbits - 1 - i
        cand_o = prefix_o | (jnp.int32(1) << bit)
        cf = _off_to_f32(cand_o)
        ones = jnp.where(noisy >= cf, jnp.int32(1), jnp.int32(0))
        cnt = jnp.sum(ones, axis=1, keepdims=True)
        newhit = (1 - hit) * jnp.where(cnt == _K, 1, 0)
        hit_cand = jnp.where(newhit == 1, cand_o, hit_cand)
        hit = hit | newhit
        prefix_o = jnp.where(cnt >= _K, cand_o, prefix_o)
        return (i + 1, prefix_o, hit, hit_cand)

    hit0 = jnp.zeros((rows, 1), jnp.int32)
    _, t_o, hit, hit_cand = jax.lax.while_loop(
        cond, body, (jnp.int32(0), prefix0, hit0, prefix0)
    )
    hf = _off_to_f32(hit_cand)
    sel_min = jnp.min(
        jnp.where(noisy >= hf, noisy, jnp.float32(jnp.inf)), axis=1, keepdims=True
    )
    tf = jnp.where(hit == 1, sel_min, _off_to_f32(t_o))
    o_ref[...] = jnp.where(noisy >= tf, x, 0.0)


@functools.partial(jax.jit)
def kernel(x):
    grid = _ROWS // _BLOCK_ROWS
    spec = pl.BlockSpec((_BLOCK_ROWS, _N), lambda i: (i, 0))
    kspec = pl.BlockSpec((_BLOCK_ROWS, 1), lambda i: (i, 0))
    return pl.pallas_call(
        _kwinners_block,
        grid=(grid,),
        in_specs=[spec, spec, kspec],
        out_specs=spec,
        out_shape=jax.ShapeDtypeStruct((_ROWS, _N), jnp.float32),
    )(x, _GUMBEL, _GUMBEL_K)
